# speedup vs baseline: 5.1615x; 5.1615x over previous
import functools

import jax
import jax.numpy as jnp
from jax import lax
from jax.experimental import pallas as pl
from jax.experimental.pallas import tpu as pltpu
from jax.experimental.pallas import tpu_sc as plsc

NC = 2
NS = 16
NW = NC * NS
CHUNK = 128


def _make_gather(n_pad, d):
    n_w = n_pad // NW
    n_chunks = n_w // CHUNK
    mesh = plsc.VectorSubcoreMesh(core_axis_name="c", subcore_axis_name="s")

    @functools.partial(
        pl.kernel,
        mesh=mesh,
        out_type=jax.ShapeDtypeStruct((n_pad, d), jnp.float32),
        scratch_types=[
            pltpu.VMEM((CHUNK,), jnp.int32),
            pltpu.VMEM((CHUNK, d), jnp.float32),
            pltpu.SemaphoreType.DMA,
        ],
    )
    def gather_kernel(table_hbm, idx_hbm, out_hbm, idx_v, rows_v, sem):
        wid = lax.axis_index("s") * NC + lax.axis_index("c")
        w_base = wid * n_w

        def body(i, carry):
            base = w_base + i * CHUNK
            pltpu.sync_copy(idx_hbm.at[pl.ds(base, CHUNK)], idx_v)
            pltpu.async_copy(table_hbm.at[idx_v], rows_v, sem).wait()
            pltpu.sync_copy(rows_v, out_hbm.at[pl.ds(base, CHUNK)])
            return carry

        lax.fori_loop(0, n_chunks, body, 0)

    return gather_kernel


def kernel(indices, W):
    b, l = indices.shape
    _, d = W.shape
    n = b * l
    idx_flat = indices.reshape(n).astype(jnp.int32)
    grain = NW * CHUNK
    n_pad = ((n + grain - 1) // grain) * grain
    if n_pad != n:
        idx_flat = jnp.pad(idx_flat, (0, n_pad - n))
    out = _make_gather(n_pad, d)(W, idx_flat)
    if n_pad != n:
        out = out[:n]
    return out.reshape(b, l, d)


# speedup vs baseline: 9.1615x; 1.7750x over previous
import functools

import jax
import jax.numpy as jnp
from jax import lax
from jax.experimental import pallas as pl
from jax.experimental.pallas import tpu as pltpu
from jax.experimental.pallas import tpu_sc as plsc

NC = 2
NS = 16
NW = NC * NS
CHUNK = 128
R = 5
LAG = 2


def _make_gather_pipelined(n_pad, d):
    n_w = n_pad // NW
    n_chunks = n_w // CHUNK
    n_groups = n_chunks // R
    mesh = plsc.VectorSubcoreMesh(core_axis_name="c", subcore_axis_name="s")

    @functools.partial(
        pl.kernel,
        mesh=mesh,
        out_type=jax.ShapeDtypeStruct((n_pad, d), jnp.float32),
        scratch_types=[
            pltpu.VMEM((n_chunks, CHUNK), jnp.int32),
            *[pltpu.VMEM((CHUNK, d), jnp.float32) for _ in range(R)],
            *[pltpu.SemaphoreType.DMA for _ in range(2 * R)],
        ],
    )
    def gather_kernel(table_hbm, idx_hbm, out_hbm, idx_v, *rest):
        rows = rest[:R]
        sg = rest[R:2 * R]
        ss = rest[2 * R:3 * R]
        wid = lax.axis_index("s") * NC + lax.axis_index("c")
        chunk0 = wid * n_chunks

        pltpu.sync_copy(idx_hbm.at[pl.ds(chunk0, n_chunks)], idx_v)

        def fire_gather(i, b):
            pltpu.async_copy(table_hbm.at[idx_v.at[i]], rows[b], sg[b])

        def wait_gather(b):
            pltpu.make_async_copy(table_hbm.at[idx_v.at[0]], rows[b], sg[b]).wait()

        def fire_store(i, b):
            dst = out_hbm.at[pl.ds((chunk0 + i) * CHUNK, CHUNK)]
            pltpu.async_copy(rows[b], dst, ss[b])

        def wait_store(b):
            dst = out_hbm.at[pl.ds(0, CHUNK)]
            pltpu.make_async_copy(rows[b], dst, ss[b]).wait()

        for i in range(R):
            fire_gather(i, i)
            if i >= LAG:
                wait_gather(i - LAG)
                fire_store(i - LAG, i - LAG)

        def body(g, carry):
            for b in range(R):
                i = g * R + b
                wait_store(b)
                fire_gather(i, b)
                bl = (b + R - LAG) % R
                wait_gather(bl)
                fire_store(i - LAG, bl)
            return carry

        lax.fori_loop(1, n_groups, body, 0)

        for i in range(n_chunks - LAG, n_chunks):
            b = i % R
            wait_gather(b)
            fire_store(i, b)
        for b in range(R):
            wait_store(b)

    return gather_kernel


def _make_gather_simple(n_pad, d):
    n_w = n_pad // NW
    n_chunks = n_w // CHUNK
    mesh = plsc.VectorSubcoreMesh(core_axis_name="c", subcore_axis_name="s")

    @functools.partial(
        pl.kernel,
        mesh=mesh,
        out_type=jax.ShapeDtypeStruct((n_pad, d), jnp.float32),
        scratch_types=[
            pltpu.VMEM((CHUNK,), jnp.int32),
            pltpu.VMEM((CHUNK, d), jnp.float32),
            pltpu.SemaphoreType.DMA,
        ],
    )
    def gather_kernel(table_hbm, idx_hbm, out_hbm, idx_v, rows_v, sem):
        wid = lax.axis_index("s") * NC + lax.axis_index("c")
        w_base = wid * n_w

        def body(i, carry):
            base = w_base + i * CHUNK
            pltpu.sync_copy(idx_hbm.at[pl.ds(base, CHUNK)], idx_v)
            pltpu.async_copy(table_hbm.at[idx_v], rows_v, sem).wait()
            pltpu.sync_copy(rows_v, out_hbm.at[pl.ds(base, CHUNK)])
            return carry

        lax.fori_loop(0, n_chunks, body, 0)

    return gather_kernel


def kernel(indices, W):
    b, l = indices.shape
    _, d = W.shape
    n = b * l
    idx_flat = indices.reshape(n).astype(jnp.int32)
    grain = NW * CHUNK * R
    n_pad = ((n + grain - 1) // grain) * grain
    if n_pad != n:
        idx_flat = jnp.pad(idx_flat, (0, n_pad - n))
    n_chunks_w = n_pad // NW // CHUNK
    if n_chunks_w // R >= 2 and n_chunks_w * CHUNK * 4 + R * CHUNK * d * 4 <= 460_000:
        out = _make_gather_pipelined(n_pad, d)(W, idx_flat.reshape(-1, CHUNK))
    else:
        out = _make_gather_simple(n_pad, d)(W, idx_flat)
    if n_pad != n:
        out = out[:n]
    return out.reshape(b, l, d)


# speedup vs baseline: 9.1902x; 1.0031x over previous
import functools

import jax
import jax.numpy as jnp
from jax import lax
from jax.experimental import pallas as pl
from jax.experimental.pallas import tpu as pltpu
from jax.experimental.pallas import tpu_sc as plsc

NC = 2
NS = 16
NW = NC * NS
CHUNK = 128
R = 5
LAG = 3


def _make_gather_pipelined(n_pad, d):
    n_w = n_pad // NW
    n_chunks = n_w // CHUNK
    n_groups = n_chunks // R
    mesh = plsc.VectorSubcoreMesh(core_axis_name="c", subcore_axis_name="s")

    @functools.partial(
        pl.kernel,
        mesh=mesh,
        out_type=jax.ShapeDtypeStruct((n_pad, d), jnp.float32),
        scratch_types=[
            pltpu.VMEM((n_chunks, CHUNK), jnp.int32),
            *[pltpu.VMEM((CHUNK, d), jnp.float32) for _ in range(R)],
            *[pltpu.SemaphoreType.DMA for _ in range(2 * R)],
        ],
    )
    def gather_kernel(table_hbm, idx_hbm, out_hbm, idx_v, *rest):
        rows = rest[:R]
        sg = rest[R:2 * R]
        ss = rest[2 * R:3 * R]
        wid = lax.axis_index("s") * NC + lax.axis_index("c")
        chunk0 = wid * n_chunks

        pltpu.sync_copy(idx_hbm.at[pl.ds(chunk0, n_chunks)], idx_v)

        def fire_gather(i, b):
            pltpu.async_copy(table_hbm.at[idx_v.at[i]], rows[b], sg[b])

        def wait_gather(b):
            pltpu.make_async_copy(table_hbm.at[idx_v.at[0]], rows[b], sg[b]).wait()

        def fire_store(i, b):
            dst = out_hbm.at[pl.ds((chunk0 + i) * CHUNK, CHUNK)]
            pltpu.async_copy(rows[b], dst, ss[b])

        def wait_store(b):
            dst = out_hbm.at[pl.ds(0, CHUNK)]
            pltpu.make_async_copy(rows[b], dst, ss[b]).wait()

        for i in range(R):
            fire_gather(i, i)
            if i >= LAG:
                wait_gather(i - LAG)
                fire_store(i - LAG, i - LAG)

        def body(g, carry):
            for b in range(R):
                i = g * R + b
                wait_store(b)
                fire_gather(i, b)
                bl = (b + R - LAG) % R
                wait_gather(bl)
                fire_store(i - LAG, bl)
            return carry

        lax.fori_loop(1, n_groups, body, 0)

        for i in range(n_chunks - LAG, n_chunks):
            b = i % R
            wait_gather(b)
            fire_store(i, b)
        for b in range(R):
            wait_store(b)

    return gather_kernel


def _make_gather_simple(n_pad, d):
    n_w = n_pad // NW
    n_chunks = n_w // CHUNK
    mesh = plsc.VectorSubcoreMesh(core_axis_name="c", subcore_axis_name="s")

    @functools.partial(
        pl.kernel,
        mesh=mesh,
        out_type=jax.ShapeDtypeStruct((n_pad, d), jnp.float32),
        scratch_types=[
            pltpu.VMEM((CHUNK,), jnp.int32),
            pltpu.VMEM((CHUNK, d), jnp.float32),
            pltpu.SemaphoreType.DMA,
        ],
    )
    def gather_kernel(table_hbm, idx_hbm, out_hbm, idx_v, rows_v, sem):
        wid = lax.axis_index("s") * NC + lax.axis_index("c")
        w_base = wid * n_w

        def body(i, carry):
            base = w_base + i * CHUNK
            pltpu.sync_copy(idx_hbm.at[pl.ds(base, CHUNK)], idx_v)
            pltpu.async_copy(table_hbm.at[idx_v], rows_v, sem).wait()
            pltpu.sync_copy(rows_v, out_hbm.at[pl.ds(base, CHUNK)])
            return carry

        lax.fori_loop(0, n_chunks, body, 0)

    return gather_kernel


def kernel(indices, W):
    b, l = indices.shape
    _, d = W.shape
    n = b * l
    idx_flat = indices.reshape(n).astype(jnp.int32)
    grain = NW * CHUNK * R
    n_pad = ((n + grain - 1) // grain) * grain
    if n_pad != n:
        idx_flat = jnp.pad(idx_flat, (0, n_pad - n))
    n_chunks_w = n_pad // NW // CHUNK
    if n_chunks_w // R >= 2 and n_chunks_w * CHUNK * 4 + R * CHUNK * d * 4 <= 460_000:
        out = _make_gather_pipelined(n_pad, d)(W, idx_flat.reshape(-1, CHUNK))
    else:
        out = _make_gather_simple(n_pad, d)(W, idx_flat)
    if n_pad != n:
        out = out[:n]
    return out.reshape(b, l, d)
